# Initial kernel scaffold; baseline (speedup 1.0000x reference)
import jax
import jax.numpy as jnp
from jax.experimental import pallas as pl


def kernel(x, W):
    raise NotImplementedError("write your pallas kernel here")


# speedup vs baseline: 1.4472x; 1.4472x over previous
import functools

import jax
import jax.numpy as jnp
from jax import lax
from jax.experimental import pallas as pl
from jax.experimental.pallas import tpu as pltpu
from jax.experimental.pallas import tpu_sc as plsc

TOKENS = 16384
DIM = 2048
NUM_EXPERTS = 64
TOP_K = 2


_TB = 1024


def _probs_body(x_ref, wt_ref, probs_ref, probs_t_ref):
    logits = jnp.dot(
        x_ref[...].astype(jnp.bfloat16),
        wt_ref[...].astype(jnp.bfloat16),
        preferred_element_type=jnp.float32,
    )
    m = jnp.max(logits, axis=-1, keepdims=True)
    e = jnp.exp(logits - m)
    p = e / jnp.sum(e, axis=-1, keepdims=True)
    probs_ref[...] = p
    probs_t_ref[...] = p.T


def _router_probs_tc(x, wt):
    return pl.pallas_call(
        _probs_body,
        grid=(TOKENS // _TB,),
        in_specs=[
            pl.BlockSpec((_TB, DIM), lambda i: (i, 0)),
            pl.BlockSpec((DIM, NUM_EXPERTS), lambda i: (0, 0)),
        ],
        out_specs=[
            pl.BlockSpec((_TB, NUM_EXPERTS), lambda i: (i, 0)),
            pl.BlockSpec((NUM_EXPERTS, _TB), lambda i: (0, i)),
        ],
        out_shape=[
            jax.ShapeDtypeStruct((TOKENS, NUM_EXPERTS), jnp.float32),
            jax.ShapeDtypeStruct((NUM_EXPERTS, TOKENS), jnp.float32),
        ],
    )(x, wt)


_NC = 2
_NS = 16
_NW = _NC * _NS
_TPW = TOKENS // _NW
_L = 16
_GROUPS = _TPW // _L


def _topk_sc_body(pt_hbm, g1_hbm, g2_hbm, i1_hbm, i2_hbm,
                  pt_v, g1_v, g2_v, i1_v, i2_v):
    wid = lax.axis_index("s") * _NC + lax.axis_index("c")
    base = wid * _TPW
    pltpu.sync_copy(pt_hbm.at[:, pl.ds(base, _TPW)], pt_v)

    def group_body(g, _):
        goff = g * _L
        max1 = jnp.full((_L,), -1.0, jnp.float32)
        max2 = jnp.full((_L,), -1.0, jnp.float32)
        idx1 = jnp.zeros((_L,), jnp.int32)
        idx2 = jnp.zeros((_L,), jnp.int32)
        for e in range(NUM_EXPERTS):
            v = pt_v[e, pl.ds(goff, _L)]
            gt1 = v > max1
            gt2 = v > max2
            max2 = jnp.where(gt1, max1, jnp.where(gt2, v, max2))
            idx2 = jnp.where(gt1, idx1, jnp.where(gt2, e, idx2))
            max1 = jnp.where(gt1, v, max1)
            idx1 = jnp.where(gt1, e, idx1)
        inv = 1.0 / (max1 + max2 + 1e-8)
        g1_v[pl.ds(goff, _L)] = max1 * inv
        g2_v[pl.ds(goff, _L)] = max2 * inv
        i1_v[pl.ds(goff, _L)] = idx1
        i2_v[pl.ds(goff, _L)] = idx2
        return 0

    lax.fori_loop(0, _GROUPS, group_body, 0)

    pltpu.sync_copy(g1_v, g1_hbm.at[pl.ds(base, _TPW)])
    pltpu.sync_copy(g2_v, g2_hbm.at[pl.ds(base, _TPW)])
    pltpu.sync_copy(i1_v, i1_hbm.at[pl.ds(base, _TPW)])
    pltpu.sync_copy(i2_v, i2_hbm.at[pl.ds(base, _TPW)])


@functools.cache
def _topk_sc():
    return pl.kernel(
        _topk_sc_body,
        mesh=plsc.VectorSubcoreMesh(core_axis_name="c", subcore_axis_name="s"),
        out_type=[
            jax.ShapeDtypeStruct((TOKENS,), jnp.float32),
            jax.ShapeDtypeStruct((TOKENS,), jnp.float32),
            jax.ShapeDtypeStruct((TOKENS,), jnp.int32),
            jax.ShapeDtypeStruct((TOKENS,), jnp.int32),
        ],
        scratch_types=[
            pltpu.VMEM((NUM_EXPERTS, _TPW), jnp.float32),
            pltpu.VMEM((_TPW,), jnp.float32),
            pltpu.VMEM((_TPW,), jnp.float32),
            pltpu.VMEM((_TPW,), jnp.int32),
            pltpu.VMEM((_TPW,), jnp.int32),
        ],
    )


def kernel(x, W):
    probs, probs_t = _router_probs_tc(x, W.T)
    g1, g2, i1, i2 = _topk_sc()(probs_t)
    gate_weights = jnp.stack([g1, g2], axis=-1)
    top_k_indices = jnp.stack([i1, i2], axis=-1)
    return (gate_weights, top_k_indices, probs)


# speedup vs baseline: 1.4574x; 1.0070x over previous
import functools

import jax
import jax.numpy as jnp
from jax import lax
from jax.experimental import pallas as pl
from jax.experimental.pallas import tpu as pltpu
from jax.experimental.pallas import tpu_sc as plsc

TOKENS = 16384
DIM = 2048
NUM_EXPERTS = 64
TOP_K = 2


_TB = 1024


def _probs_body(x_ref, wt_ref, probs_ref, probs_t_ref):
    logits = jnp.dot(
        x_ref[...].astype(jnp.bfloat16),
        wt_ref[...].astype(jnp.bfloat16),
        preferred_element_type=jnp.float32,
    )
    m = jnp.max(logits, axis=-1, keepdims=True)
    e = jnp.exp(logits - m)
    p = e / jnp.sum(e, axis=-1, keepdims=True)
    probs_ref[...] = p
    probs_t_ref[...] = p.T


def _router_probs_tc(x, wt):
    return pl.pallas_call(
        _probs_body,
        grid=(TOKENS // _TB,),
        in_specs=[
            pl.BlockSpec((_TB, DIM), lambda i: (i, 0)),
            pl.BlockSpec((DIM, NUM_EXPERTS), lambda i: (0, 0)),
        ],
        out_specs=[
            pl.BlockSpec((_TB, NUM_EXPERTS), lambda i: (i, 0)),
            pl.BlockSpec((NUM_EXPERTS, _TB), lambda i: (0, i)),
        ],
        out_shape=[
            jax.ShapeDtypeStruct((TOKENS, NUM_EXPERTS), jnp.float32),
            jax.ShapeDtypeStruct((NUM_EXPERTS, TOKENS), jnp.float32),
        ],
    )(x, wt)


_NC = 2
_NS = 16
_NW = _NC * _NS
_TPW = TOKENS // _NW
_L = 16
_GROUPS = _TPW // _L


def _topk_sc_body(pt_hbm, g1_hbm, g2_hbm, i1_hbm, i2_hbm,
                  pt_v, g1_v, g2_v, i1_v, i2_v):
    wid = lax.axis_index("s") * _NC + lax.axis_index("c")
    base = wid * _TPW
    pltpu.sync_copy(pt_hbm.at[:, pl.ds(base, _TPW)], pt_v)

    def chain(goff, e_lo, e_hi):
        max1 = jnp.full((_L,), -1.0, jnp.float32)
        max2 = jnp.full((_L,), -1.0, jnp.float32)
        idx1 = jnp.zeros((_L,), jnp.int32)
        idx2 = jnp.zeros((_L,), jnp.int32)
        for e in range(e_lo, e_hi):
            v = pt_v[e, pl.ds(goff, _L)]
            gt1 = v > max1
            gt2 = v > max2
            max2 = jnp.where(gt1, max1, jnp.where(gt2, v, max2))
            idx2 = jnp.where(gt1, idx1, jnp.where(gt2, e, idx2))
            max1 = jnp.where(gt1, v, max1)
            idx1 = jnp.where(gt1, e, idx1)
        return max1, idx1, max2, idx2

    def merge(a, b):
        a1, ai1, a2, ai2 = a
        b1, bi1, b2, bi2 = b
        gt = b1 > a1
        n1 = jnp.where(gt, b1, a1)
        ni1 = jnp.where(gt, bi1, ai1)
        sg = b2 > a1
        sec_gt = jnp.where(sg, b2, a1)
        seci_gt = jnp.where(sg, bi2, ai1)
        sn = a2 >= b1
        sec_ng = jnp.where(sn, a2, b1)
        seci_ng = jnp.where(sn, ai2, bi1)
        n2 = jnp.where(gt, sec_gt, sec_ng)
        ni2 = jnp.where(gt, seci_gt, seci_ng)
        return n1, ni1, n2, ni2

    def group_body(g, _):
        goff = g * _L
        c0 = chain(goff, 0, 16)
        c1 = chain(goff, 16, 32)
        c2 = chain(goff, 32, 48)
        c3 = chain(goff, 48, 64)
        max1, idx1, max2, idx2 = merge(merge(c0, c1), merge(c2, c3))
        inv = 1.0 / (max1 + max2 + 1e-8)
        g1_v[pl.ds(goff, _L)] = max1 * inv
        g2_v[pl.ds(goff, _L)] = max2 * inv
        i1_v[pl.ds(goff, _L)] = idx1
        i2_v[pl.ds(goff, _L)] = idx2
        return 0

    lax.fori_loop(0, _GROUPS, group_body, 0)

    pltpu.sync_copy(g1_v, g1_hbm.at[pl.ds(base, _TPW)])
    pltpu.sync_copy(g2_v, g2_hbm.at[pl.ds(base, _TPW)])
    pltpu.sync_copy(i1_v, i1_hbm.at[pl.ds(base, _TPW)])
    pltpu.sync_copy(i2_v, i2_hbm.at[pl.ds(base, _TPW)])


@functools.cache
def _topk_sc():
    return pl.kernel(
        _topk_sc_body,
        mesh=plsc.VectorSubcoreMesh(core_axis_name="c", subcore_axis_name="s"),
        out_type=[
            jax.ShapeDtypeStruct((TOKENS,), jnp.float32),
            jax.ShapeDtypeStruct((TOKENS,), jnp.float32),
            jax.ShapeDtypeStruct((TOKENS,), jnp.int32),
            jax.ShapeDtypeStruct((TOKENS,), jnp.int32),
        ],
        scratch_types=[
            pltpu.VMEM((NUM_EXPERTS, _TPW), jnp.float32),
            pltpu.VMEM((_TPW,), jnp.float32),
            pltpu.VMEM((_TPW,), jnp.float32),
            pltpu.VMEM((_TPW,), jnp.int32),
            pltpu.VMEM((_TPW,), jnp.int32),
        ],
    )


def kernel(x, W):
    probs, probs_t = _router_probs_tc(x, W.T)
    g1, g2, i1, i2 = _topk_sc()(probs_t)
    gate_weights = jnp.stack([g1, g2], axis=-1)
    top_k_indices = jnp.stack([i1, i2], axis=-1)
    return (gate_weights, top_k_indices, probs)


# speedup vs baseline: 1.5081x; 1.0348x over previous
import functools

import jax
import jax.numpy as jnp
from jax import lax
from jax.experimental import pallas as pl
from jax.experimental.pallas import tpu as pltpu
from jax.experimental.pallas import tpu_sc as plsc

TOKENS = 16384
DIM = 2048
NUM_EXPERTS = 64
TOP_K = 2


_TB = 1024


def _probs_body(x_ref, w_ref, probs_ref, probs_t_ref):
    logits = lax.dot_general(
        x_ref[...].astype(jnp.bfloat16),
        w_ref[...].astype(jnp.bfloat16),
        dimension_numbers=(((1,), (1,)), ((), ())),
        preferred_element_type=jnp.float32,
    )
    m = jnp.max(logits, axis=-1, keepdims=True)
    e = jnp.exp(logits - m)
    p = e / jnp.sum(e, axis=-1, keepdims=True)
    probs_ref[...] = p
    probs_t_ref[...] = p.T


def _router_probs_tc(x, w):
    return pl.pallas_call(
        _probs_body,
        grid=(TOKENS // _TB,),
        in_specs=[
            pl.BlockSpec((_TB, DIM), lambda i: (i, 0)),
            pl.BlockSpec((NUM_EXPERTS, DIM), lambda i: (0, 0)),
        ],
        out_specs=[
            pl.BlockSpec((_TB, NUM_EXPERTS), lambda i: (i, 0)),
            pl.BlockSpec((NUM_EXPERTS, _TB), lambda i: (0, i)),
        ],
        out_shape=[
            jax.ShapeDtypeStruct((TOKENS, NUM_EXPERTS), jnp.float32),
            jax.ShapeDtypeStruct((NUM_EXPERTS, TOKENS), jnp.float32),
        ],
    )(x, w)


_NC = 2
_NS = 16
_NW = _NC * _NS
_TPW = TOKENS // _NW
_L = 16
_GROUPS = _TPW // _L


def _topk_sc_body(pt_hbm, g1_hbm, g2_hbm, i1_hbm, i2_hbm,
                  pt_v, g1_v, g2_v, i1_v, i2_v):
    wid = lax.axis_index("s") * _NC + lax.axis_index("c")
    base = wid * _TPW
    pltpu.sync_copy(pt_hbm.at[:, pl.ds(base, _TPW)], pt_v)

    def chain(goff, e_lo, e_hi):
        max1 = jnp.full((_L,), -1.0, jnp.float32)
        max2 = jnp.full((_L,), -1.0, jnp.float32)
        idx1 = jnp.zeros((_L,), jnp.int32)
        idx2 = jnp.zeros((_L,), jnp.int32)
        for e in range(e_lo, e_hi):
            v = pt_v[e, pl.ds(goff, _L)]
            gt1 = v > max1
            gt2 = v > max2
            max2 = jnp.where(gt1, max1, jnp.where(gt2, v, max2))
            idx2 = jnp.where(gt1, idx1, jnp.where(gt2, e, idx2))
            max1 = jnp.where(gt1, v, max1)
            idx1 = jnp.where(gt1, e, idx1)
        return max1, idx1, max2, idx2

    def merge(a, b):
        a1, ai1, a2, ai2 = a
        b1, bi1, b2, bi2 = b
        gt = b1 > a1
        n1 = jnp.where(gt, b1, a1)
        ni1 = jnp.where(gt, bi1, ai1)
        sg = b2 > a1
        sec_gt = jnp.where(sg, b2, a1)
        seci_gt = jnp.where(sg, bi2, ai1)
        sn = a2 >= b1
        sec_ng = jnp.where(sn, a2, b1)
        seci_ng = jnp.where(sn, ai2, bi1)
        n2 = jnp.where(gt, sec_gt, sec_ng)
        ni2 = jnp.where(gt, seci_gt, seci_ng)
        return n1, ni1, n2, ni2

    def group_body(g, _):
        goff = g * _L
        c0 = chain(goff, 0, 16)
        c1 = chain(goff, 16, 32)
        c2 = chain(goff, 32, 48)
        c3 = chain(goff, 48, 64)
        max1, idx1, max2, idx2 = merge(merge(c0, c1), merge(c2, c3))
        inv = 1.0 / (max1 + max2 + 1e-8)
        g1_v[pl.ds(goff, _L)] = max1 * inv
        g2_v[pl.ds(goff, _L)] = max2 * inv
        i1_v[pl.ds(goff, _L)] = idx1
        i2_v[pl.ds(goff, _L)] = idx2
        return 0

    lax.fori_loop(0, _GROUPS, group_body, 0)

    pltpu.sync_copy(g1_v, g1_hbm.at[pl.ds(base, _TPW)])
    pltpu.sync_copy(g2_v, g2_hbm.at[pl.ds(base, _TPW)])
    pltpu.sync_copy(i1_v, i1_hbm.at[pl.ds(base, _TPW)])
    pltpu.sync_copy(i2_v, i2_hbm.at[pl.ds(base, _TPW)])


@functools.cache
def _topk_sc():
    return pl.kernel(
        _topk_sc_body,
        mesh=plsc.VectorSubcoreMesh(core_axis_name="c", subcore_axis_name="s"),
        out_type=[
            jax.ShapeDtypeStruct((TOKENS,), jnp.float32),
            jax.ShapeDtypeStruct((TOKENS,), jnp.float32),
            jax.ShapeDtypeStruct((TOKENS,), jnp.int32),
            jax.ShapeDtypeStruct((TOKENS,), jnp.int32),
        ],
        scratch_types=[
            pltpu.VMEM((NUM_EXPERTS, _TPW), jnp.float32),
            pltpu.VMEM((_TPW,), jnp.float32),
            pltpu.VMEM((_TPW,), jnp.float32),
            pltpu.VMEM((_TPW,), jnp.int32),
            pltpu.VMEM((_TPW,), jnp.int32),
        ],
    )


def kernel(x, W):
    probs, probs_t = _router_probs_tc(x, W)
    g1, g2, i1, i2 = _topk_sc()(probs_t)
    gate_weights = jnp.stack([g1, g2], axis=-1)
    top_k_indices = jnp.stack([i1, i2], axis=-1)
    return (gate_weights, top_k_indices, probs)


# speedup vs baseline: 1.5291x; 1.0139x over previous
import functools

import jax
import jax.numpy as jnp
from jax import lax
from jax.experimental import pallas as pl
from jax.experimental.pallas import tpu as pltpu
from jax.experimental.pallas import tpu_sc as plsc

TOKENS = 16384
DIM = 2048
NUM_EXPERTS = 64
TOP_K = 2


_TB = 2048


def _probs_body(x_ref, w_ref, probs_ref, probs_t_ref):
    logits = lax.dot_general(
        x_ref[...].astype(jnp.bfloat16),
        w_ref[...].astype(jnp.bfloat16),
        dimension_numbers=(((1,), (1,)), ((), ())),
        preferred_element_type=jnp.float32,
    )
    m = jnp.max(logits, axis=-1, keepdims=True)
    e = jnp.exp(logits - m)
    p = e / jnp.sum(e, axis=-1, keepdims=True)
    probs_ref[...] = p
    probs_t_ref[...] = p.T


def _router_probs_tc(x, w):
    return pl.pallas_call(
        _probs_body,
        grid=(TOKENS // _TB,),
        in_specs=[
            pl.BlockSpec((_TB, DIM), lambda i: (i, 0)),
            pl.BlockSpec((NUM_EXPERTS, DIM), lambda i: (0, 0)),
        ],
        out_specs=[
            pl.BlockSpec((_TB, NUM_EXPERTS), lambda i: (i, 0)),
            pl.BlockSpec((NUM_EXPERTS, _TB), lambda i: (0, i)),
        ],
        out_shape=[
            jax.ShapeDtypeStruct((TOKENS, NUM_EXPERTS), jnp.float32),
            jax.ShapeDtypeStruct((NUM_EXPERTS, TOKENS), jnp.float32),
        ],
    )(x, w)


_NC = 2
_NS = 16
_NW = _NC * _NS
_TPW = TOKENS // _NW
_L = 16
_GROUPS = _TPW // _L


def _topk_sc_body(pt_hbm, g1_hbm, g2_hbm, i1_hbm, i2_hbm,
                  pt_v, g1_v, g2_v, i1_v, i2_v):
    wid = lax.axis_index("s") * _NC + lax.axis_index("c")
    base = wid * _TPW
    pltpu.sync_copy(pt_hbm.at[:, pl.ds(base, _TPW)], pt_v)

    def chain(goff, e_lo, e_hi):
        max1 = jnp.full((_L,), -1.0, jnp.float32)
        max2 = jnp.full((_L,), -1.0, jnp.float32)
        idx1 = jnp.zeros((_L,), jnp.int32)
        idx2 = jnp.zeros((_L,), jnp.int32)
        for e in range(e_lo, e_hi):
            v = pt_v[e, pl.ds(goff, _L)]
            gt1 = v > max1
            gt2 = v > max2
            max2 = jnp.where(gt1, max1, jnp.where(gt2, v, max2))
            idx2 = jnp.where(gt1, idx1, jnp.where(gt2, e, idx2))
            max1 = jnp.where(gt1, v, max1)
            idx1 = jnp.where(gt1, e, idx1)
        return max1, idx1, max2, idx2

    def merge(a, b):
        a1, ai1, a2, ai2 = a
        b1, bi1, b2, bi2 = b
        gt = b1 > a1
        n1 = jnp.where(gt, b1, a1)
        ni1 = jnp.where(gt, bi1, ai1)
        sg = b2 > a1
        sec_gt = jnp.where(sg, b2, a1)
        seci_gt = jnp.where(sg, bi2, ai1)
        sn = a2 >= b1
        sec_ng = jnp.where(sn, a2, b1)
        seci_ng = jnp.where(sn, ai2, bi1)
        n2 = jnp.where(gt, sec_gt, sec_ng)
        ni2 = jnp.where(gt, seci_gt, seci_ng)
        return n1, ni1, n2, ni2

    def group_body(g, _):
        goff = g * _L
        c0 = chain(goff, 0, 16)
        c1 = chain(goff, 16, 32)
        c2 = chain(goff, 32, 48)
        c3 = chain(goff, 48, 64)
        max1, idx1, max2, idx2 = merge(merge(c0, c1), merge(c2, c3))
        inv = 1.0 / (max1 + max2 + 1e-8)
        g1_v[pl.ds(goff, _L)] = max1 * inv
        g2_v[pl.ds(goff, _L)] = max2 * inv
        i1_v[pl.ds(goff, _L)] = idx1
        i2_v[pl.ds(goff, _L)] = idx2
        return 0

    lax.fori_loop(0, _GROUPS, group_body, 0)

    pltpu.sync_copy(g1_v, g1_hbm.at[pl.ds(base, _TPW)])
    pltpu.sync_copy(g2_v, g2_hbm.at[pl.ds(base, _TPW)])
    pltpu.sync_copy(i1_v, i1_hbm.at[pl.ds(base, _TPW)])
    pltpu.sync_copy(i2_v, i2_hbm.at[pl.ds(base, _TPW)])


@functools.cache
def _topk_sc():
    return pl.kernel(
        _topk_sc_body,
        mesh=plsc.VectorSubcoreMesh(core_axis_name="c", subcore_axis_name="s"),
        out_type=[
            jax.ShapeDtypeStruct((TOKENS,), jnp.float32),
            jax.ShapeDtypeStruct((TOKENS,), jnp.float32),
            jax.ShapeDtypeStruct((TOKENS,), jnp.int32),
            jax.ShapeDtypeStruct((TOKENS,), jnp.int32),
        ],
        scratch_types=[
            pltpu.VMEM((NUM_EXPERTS, _TPW), jnp.float32),
            pltpu.VMEM((_TPW,), jnp.float32),
            pltpu.VMEM((_TPW,), jnp.float32),
            pltpu.VMEM((_TPW,), jnp.int32),
            pltpu.VMEM((_TPW,), jnp.int32),
        ],
    )


def kernel(x, W):
    probs, probs_t = _router_probs_tc(x, W)
    g1, g2, i1, i2 = _topk_sc()(probs_t)
    gate_weights = jnp.stack([g1, g2], axis=-1)
    top_k_indices = jnp.stack([i1, i2], axis=-1)
    return (gate_weights, top_k_indices, probs)
